# baseline (speedup 1.0000x reference)
import functools

import jax
import jax.numpy as jnp
from jax import lax
from jax.experimental import pallas as pl
from jax.experimental.pallas import tpu as pltpu
from jax.experimental.pallas import tpu_sc as plsc

N = 10000
D = 128
NC, NS, L = 2, 16, 16
NW = NC * NS
SB = 32
W = 30
NB = 10
N_PAD = 10240
RPT = N_PAD // NS
TAIL = N - (NS - 1) * RPT


def _sc_agg_body(with_deg, n_windows, *refs):
    if with_deg:
        (x_hbm, cols_hbm, rows_hbm, z2d_hbm, z1d_hbm, part_hbm, deg_hbm,
         colsv, rowsv, gbuf, onesv, acc, dacc,
         isem, zsem, gsem, ssem) = refs
    else:
        (x_hbm, cols_hbm, rows_hbm, z2d_hbm, part_hbm,
         colsv, rowsv, gbuf, acc, isem, zsem, gsem, ssem) = refs

    c = lax.axis_index("c")
    s = lax.axis_index("s")
    wid = s * NC + c
    row0 = s * RPT

    def stage(win, slot):
        pltpu.async_copy(cols_hbm.at[wid, win], colsv.at[slot],
                         isem.at[slot])
        pltpu.async_copy(rows_hbm.at[wid, win], rowsv.at[slot],
                         isem.at[slot])

    def stage_wait(win, slot):
        pltpu.make_async_copy(cols_hbm.at[wid, win], colsv.at[slot],
                              isem.at[slot]).wait()
        pltpu.make_async_copy(rows_hbm.at[wid, win], rowsv.at[slot],
                              isem.at[slot]).wait()

    stage(0, 0)

    @pl.when(n_windows > 1)
    def _stage1():
        stage(1, 1)

    pltpu.async_copy(z2d_hbm, acc.at[pl.ds(row0, RPT)], zsem)
    if with_deg:
        pltpu.async_copy(z1d_hbm, dacc.at[pl.ds(row0, RPT)], zsem)
        ones16 = jnp.ones((L,), jnp.float32)
        for i in range(SB // L):
            onesv[pl.ds(i * L, L)] = ones16

    pltpu.make_async_copy(z2d_hbm, acc.at[pl.ds(row0, RPT)], zsem).wait()
    if with_deg:
        pltpu.make_async_copy(z1d_hbm, dacc.at[pl.ds(row0, RPT)],
                              zsem).wait()

    plsc.subcore_barrier()

    n_total = n_windows * W
    stage_wait(0, 0)
    for u0 in range(NB - 1):
        pltpu.async_copy(x_hbm.at[colsv.at[0, u0]], gbuf.at[u0],
                         gsem.at[u0])

    def window(w, carry):
        p = lax.rem(w, 3)
        p1 = lax.rem(w + 1, 3)
        p2 = lax.rem(w + 2, 3)
        for u in range(W):
            b = u % NB
            bg = (u + NB - 1) % NB
            j = w * W + u
            pltpu.make_async_copy(x_hbm.at[colsv.at[p, u]], gbuf.at[b],
                                  gsem.at[b]).wait()
            pltpu.async_copy(gbuf.at[b], acc.at[rowsv.at[p, u]],
                             ssem.at[b], add=True)
            if with_deg:
                pltpu.sync_copy(onesv, dacc.at[rowsv.at[p, u]], add=True)
            if u == 2:
                @pl.when(w + 2 < n_windows)
                def _prefetch():
                    stage(w + 2, p2)
            if u == W - NB:
                @pl.when(w + 1 < n_windows)
                def _wait_next_idx():
                    stage_wait(w + 1, p1)

            @pl.when(j >= 1)
            def _drain():
                pltpu.make_async_copy(gbuf.at[bg], acc.at[rowsv.at[p, u]],
                                      ssem.at[bg]).wait()

            @pl.when(j + NB - 1 < n_total)
            def _issue():
                if u + NB - 1 < W:
                    pltpu.async_copy(x_hbm.at[colsv.at[p, u + NB - 1]],
                                     gbuf.at[bg], gsem.at[bg])
                else:
                    pltpu.async_copy(x_hbm.at[colsv.at[p1, u + NB - 1 - W]],
                                     gbuf.at[bg], gsem.at[bg])
        return carry
    lax.fori_loop(0, n_windows, window, None)

    pltpu.make_async_copy(gbuf.at[(n_total - 1) % NB],
                          acc.at[rowsv.at[0, 0]],
                          ssem.at[(n_total - 1) % NB]).wait()

    plsc.subcore_barrier()

    @pl.when(s < NS - 1)
    def _wb_full():
        pltpu.sync_copy(acc.at[pl.ds(row0, RPT)],
                        part_hbm.at[c, pl.ds(row0, RPT)])
        if with_deg:
            pltpu.sync_copy(dacc.at[pl.ds(row0, RPT)],
                            deg_hbm.at[c, pl.ds(row0, RPT)])

    @pl.when(s == NS - 1)
    def _wb_tail():
        pltpu.sync_copy(acc.at[pl.ds(row0, TAIL)],
                        part_hbm.at[c, pl.ds(row0, TAIL)])
        if with_deg:
            pltpu.sync_copy(dacc.at[pl.ds(row0, TAIL)],
                            deg_hbm.at[c, pl.ds(row0, TAIL)])


def _make_sc_agg(n_windows, with_deg):
    mesh = plsc.VectorSubcoreMesh(core_axis_name="c", subcore_axis_name="s",
                                  num_cores=NC, num_subcores=NS)
    out_type = [jax.ShapeDtypeStruct((NC, N, D), jnp.float32)]
    scratch = [
        pltpu.VMEM((3, W, SB), jnp.int32),
        pltpu.VMEM((3, W, SB), jnp.int32),
        pltpu.VMEM((NB, SB, D), jnp.float32),
    ]
    if with_deg:
        out_type.append(jax.ShapeDtypeStruct((NC, N), jnp.float32))
        scratch.append(pltpu.VMEM((SB,), jnp.float32))
    scratch.append(pltpu.VMEM_SHARED((N_PAD, D), jnp.float32))
    if with_deg:
        scratch.append(pltpu.VMEM_SHARED((N_PAD,), jnp.float32))
    scratch += [pltpu.SemaphoreType.DMA((3,)),
                pltpu.SemaphoreType.DMA,
                pltpu.SemaphoreType.DMA((NB,)),
                pltpu.SemaphoreType.DMA((NB,))]
    return pl.kernel(
        functools.partial(_sc_agg_body, with_deg, n_windows),
        out_type=tuple(out_type),
        mesh=mesh,
        scratch_types=scratch,
        compiler_params=pltpu.CompilerParams(use_tc_tiling_on_sc=False),
    )


def _dense_body(x_ref, p0_ref, p1_ref, dt_ref,
                wt_ref, wb_ref, b_ref, o_ref):
    inv = 1.0 / jnp.maximum(dt_ref[:, 0:1] + dt_ref[:, 1:2], 1.0)
    neigh = (p0_ref[0] + p1_ref[0]) * inv
    acc = jnp.dot(x_ref[...], wt_ref[...], preferred_element_type=jnp.float32)
    acc = acc + jnp.dot(neigh, wb_ref[...],
                        preferred_element_type=jnp.float32)
    o_ref[...] = jnp.maximum(acc + b_ref[...], 0.0)


def _dense(x, part, degt, wt, wb, b):
    R = 1000
    grid = (N // R,)
    return pl.pallas_call(
        _dense_body,
        grid=grid,
        in_specs=[
            pl.BlockSpec((R, D), lambda i: (i, 0)),
            pl.BlockSpec((1, R, D), lambda i: (0, i, 0)),
            pl.BlockSpec((1, R, D), lambda i: (1, i, 0)),
            pl.BlockSpec((R, 2), lambda i: (i, 0)),
            pl.BlockSpec((D, D), lambda i: (0, 0)),
            pl.BlockSpec((D, D), lambda i: (0, 0)),
            pl.BlockSpec((1, D), lambda i: (0, 0)),
        ],
        out_specs=pl.BlockSpec((R, D), lambda i: (i, 0)),
        out_shape=jax.ShapeDtypeStruct((N, D), jnp.float32),
    )(x, part, part, degt, wt, wb, b)


def kernel(features, rows, cols, W1, b1, W2, b2):
    E = rows.shape[0]
    n_windows = -(-E // (NW * SB * W))
    e_pad = NW * n_windows * W * SB

    pad = e_pad - E
    i = jnp.arange(pad, dtype=jnp.int32)
    cols_p = jnp.concatenate([cols, i % N])
    rows_p = jnp.concatenate([rows, N + i % (N_PAD - N)])
    cols_r = cols_p.reshape(NW, n_windows, W, SB)
    rows_r = rows_p.reshape(NW, n_windows, W, SB)

    agg1 = _make_sc_agg(n_windows, with_deg=True)
    agg2 = _make_sc_agg(n_windows, with_deg=False)

    z2d = jnp.zeros((RPT, D), jnp.float32)
    z1d = jnp.zeros((RPT,), jnp.float32)
    part1, degp = agg1(features, cols_r, rows_r, z2d, z1d)
    degt = degp.T
    w1t, w1b = W1[:D], W1[D:]
    w2t, w2b = W2[:D], W2[D:]

    h1 = _dense(features, part1, degt, w1t, w1b, b1[None, :])
    (part2,) = agg2(h1, cols_r, rows_r, z2d)
    h2 = _dense(h1, part2, degt, w2t, w2b, b2[None, :])
    return h2


# speedup vs baseline: 1.0260x; 1.0260x over previous
import functools

import jax
import jax.numpy as jnp
from jax import lax
from jax.experimental import pallas as pl
from jax.experimental.pallas import tpu as pltpu
from jax.experimental.pallas import tpu_sc as plsc

N = 10000
D = 128
NC, NS, L = 2, 16, 16
NW = NC * NS
SB = 32
W = 32
NB = 8
N_PAD = 10240
RPT = N_PAD // NS
TAIL = N - (NS - 1) * RPT


def _sc_agg_body(with_deg, n_windows, *refs):
    if with_deg:
        (x_hbm, cols_hbm, rows_hbm, z2d_hbm, z1d_hbm, part_hbm, deg_hbm,
         colsv, rowsv, gbuf, onesv, acc, dacc,
         isem, zsem, gsem, ssem) = refs
    else:
        (x_hbm, cols_hbm, rows_hbm, z2d_hbm, part_hbm,
         colsv, rowsv, gbuf, acc, isem, zsem, gsem, ssem) = refs

    c = lax.axis_index("c")
    s = lax.axis_index("s")
    wid = s * NC + c
    row0 = s * RPT

    def stage(win, slot):
        pltpu.async_copy(cols_hbm.at[wid, win], colsv.at[slot],
                         isem.at[slot])
        pltpu.async_copy(rows_hbm.at[wid, win], rowsv.at[slot],
                         isem.at[slot])

    def stage_wait(win, slot):
        pltpu.make_async_copy(cols_hbm.at[wid, win], colsv.at[slot],
                              isem.at[slot]).wait()
        pltpu.make_async_copy(rows_hbm.at[wid, win], rowsv.at[slot],
                              isem.at[slot]).wait()

    stage(0, 0)

    @pl.when(n_windows > 1)
    def _stage1():
        stage(1, 1)

    pltpu.async_copy(z2d_hbm, acc.at[pl.ds(row0, RPT)], zsem)
    if with_deg:
        pltpu.async_copy(z1d_hbm, dacc.at[pl.ds(row0, RPT)], zsem)
        ones16 = jnp.ones((L,), jnp.float32)
        for i in range(SB // L):
            onesv[pl.ds(i * L, L)] = ones16

    pltpu.make_async_copy(z2d_hbm, acc.at[pl.ds(row0, RPT)], zsem).wait()
    if with_deg:
        pltpu.make_async_copy(z1d_hbm, dacc.at[pl.ds(row0, RPT)],
                              zsem).wait()

    plsc.subcore_barrier()

    n_total = n_windows * W
    stage_wait(0, 0)
    for u0 in range(NB - 1):
        pltpu.async_copy(x_hbm.at[colsv.at[0, u0]], gbuf.at[u0],
                         gsem.at[u0])

    def window(w, carry):
        p = lax.rem(w, 3)
        p1 = lax.rem(w + 1, 3)
        p2 = lax.rem(w + 2, 3)
        for u in range(W):
            b = u % NB
            bg = (u + NB - 1) % NB
            j = w * W + u
            pltpu.make_async_copy(x_hbm.at[colsv.at[p, u]], gbuf.at[b],
                                  gsem.at[b]).wait()
            pltpu.async_copy(gbuf.at[b], acc.at[rowsv.at[p, u]],
                             ssem.at[b], add=True)
            if with_deg:
                pltpu.sync_copy(onesv, dacc.at[rowsv.at[p, u]], add=True)
            if u == 2:
                @pl.when(w + 2 < n_windows)
                def _prefetch():
                    stage(w + 2, p2)
            if u == W - NB:
                @pl.when(w + 1 < n_windows)
                def _wait_next_idx():
                    stage_wait(w + 1, p1)

            @pl.when(j >= 1)
            def _drain():
                pltpu.make_async_copy(gbuf.at[bg], acc.at[rowsv.at[p, u]],
                                      ssem.at[bg]).wait()

            @pl.when(j + NB - 1 < n_total)
            def _issue():
                if u + NB - 1 < W:
                    pltpu.async_copy(x_hbm.at[colsv.at[p, u + NB - 1]],
                                     gbuf.at[bg], gsem.at[bg])
                else:
                    pltpu.async_copy(x_hbm.at[colsv.at[p1, u + NB - 1 - W]],
                                     gbuf.at[bg], gsem.at[bg])
        return carry
    lax.fori_loop(0, n_windows, window, None)

    pltpu.make_async_copy(gbuf.at[(n_total - 1) % NB],
                          acc.at[rowsv.at[0, 0]],
                          ssem.at[(n_total - 1) % NB]).wait()

    plsc.subcore_barrier()

    @pl.when(s < NS - 1)
    def _wb_full():
        pltpu.sync_copy(acc.at[pl.ds(row0, RPT)],
                        part_hbm.at[c, pl.ds(row0, RPT)])
        if with_deg:
            pltpu.sync_copy(dacc.at[pl.ds(row0, RPT)],
                            deg_hbm.at[c, pl.ds(row0, RPT)])

    @pl.when(s == NS - 1)
    def _wb_tail():
        pltpu.sync_copy(acc.at[pl.ds(row0, TAIL)],
                        part_hbm.at[c, pl.ds(row0, TAIL)])
        if with_deg:
            pltpu.sync_copy(dacc.at[pl.ds(row0, TAIL)],
                            deg_hbm.at[c, pl.ds(row0, TAIL)])


def _make_sc_agg(n_windows, with_deg):
    mesh = plsc.VectorSubcoreMesh(core_axis_name="c", subcore_axis_name="s",
                                  num_cores=NC, num_subcores=NS)
    out_type = [jax.ShapeDtypeStruct((NC, N, D), jnp.float32)]
    scratch = [
        pltpu.VMEM((3, W, SB), jnp.int32),
        pltpu.VMEM((3, W, SB), jnp.int32),
        pltpu.VMEM((NB, SB, D), jnp.float32),
    ]
    if with_deg:
        out_type.append(jax.ShapeDtypeStruct((NC, N), jnp.float32))
        scratch.append(pltpu.VMEM((SB,), jnp.float32))
    scratch.append(pltpu.VMEM_SHARED((N_PAD, D), jnp.float32))
    if with_deg:
        scratch.append(pltpu.VMEM_SHARED((N_PAD,), jnp.float32))
    scratch += [pltpu.SemaphoreType.DMA((3,)),
                pltpu.SemaphoreType.DMA,
                pltpu.SemaphoreType.DMA((NB,)),
                pltpu.SemaphoreType.DMA((NB,))]
    return pl.kernel(
        functools.partial(_sc_agg_body, with_deg, n_windows),
        out_type=tuple(out_type),
        mesh=mesh,
        scratch_types=scratch,
        compiler_params=pltpu.CompilerParams(use_tc_tiling_on_sc=False),
    )


def _dense_body(x_ref, p0_ref, p1_ref, dt_ref,
                wt_ref, wb_ref, b_ref, o_ref):
    inv = 1.0 / jnp.maximum(dt_ref[:, 0:1] + dt_ref[:, 1:2], 1.0)
    neigh = (p0_ref[0] + p1_ref[0]) * inv
    acc = jnp.dot(x_ref[...], wt_ref[...], preferred_element_type=jnp.float32)
    acc = acc + jnp.dot(neigh, wb_ref[...],
                        preferred_element_type=jnp.float32)
    o_ref[...] = jnp.maximum(acc + b_ref[...], 0.0)


def _dense(x, part, degt, wt, wb, b):
    R = 1000
    grid = (N // R,)
    return pl.pallas_call(
        _dense_body,
        grid=grid,
        in_specs=[
            pl.BlockSpec((R, D), lambda i: (i, 0)),
            pl.BlockSpec((1, R, D), lambda i: (0, i, 0)),
            pl.BlockSpec((1, R, D), lambda i: (1, i, 0)),
            pl.BlockSpec((R, 2), lambda i: (i, 0)),
            pl.BlockSpec((D, D), lambda i: (0, 0)),
            pl.BlockSpec((D, D), lambda i: (0, 0)),
            pl.BlockSpec((1, D), lambda i: (0, 0)),
        ],
        out_specs=pl.BlockSpec((R, D), lambda i: (i, 0)),
        out_shape=jax.ShapeDtypeStruct((N, D), jnp.float32),
    )(x, part, part, degt, wt, wb, b)


def kernel(features, rows, cols, W1, b1, W2, b2):
    E = rows.shape[0]
    n_windows = -(-E // (NW * SB * W))
    e_pad = NW * n_windows * W * SB

    pad = e_pad - E
    i = jnp.arange(pad, dtype=jnp.int32)
    cols_p = jnp.concatenate([cols, i % N])
    rows_p = jnp.concatenate([rows, N + i % (N_PAD - N)])
    cols_r = cols_p.reshape(NW, n_windows, W, SB)
    rows_r = rows_p.reshape(NW, n_windows, W, SB)

    agg1 = _make_sc_agg(n_windows, with_deg=True)
    agg2 = _make_sc_agg(n_windows, with_deg=False)

    z2d = jnp.zeros((RPT, D), jnp.float32)
    z1d = jnp.zeros((RPT,), jnp.float32)
    part1, degp = agg1(features, cols_r, rows_r, z2d, z1d)
    degt = degp.T
    w1t, w1b = W1[:D], W1[D:]
    w2t, w2b = W2[:D], W2[D:]

    h1 = _dense(features, part1, degt, w1t, w1b, b1[None, :])
    (part2,) = agg2(h1, cols_r, rows_r, z2d)
    h2 = _dense(h1, part2, degt, w2t, w2b, b2[None, :])
    return h2
